# speedup vs baseline: 10.0137x; 10.0137x over previous
import functools

import jax
import jax.numpy as jnp
from jax import lax
from jax.experimental import pallas as pl
from jax.experimental.pallas import tpu as pltpu
from jax.experimental.pallas import tpu_sc as plsc

N = 10000
E = 320000
D = 128

NPAD = 10240
C = 128
NW = 32
CHUNKS = 79
EPAD = NW * CHUNKS * C
STRIPE = NPAD // 16

_mesh = plsc.VectorSubcoreMesh(core_axis_name="c", subcore_axis_name="s")


@functools.partial(
    pl.kernel,
    out_type=jax.ShapeDtypeStruct((2, NPAD), jnp.float32),
    mesh=_mesh,
    scratch_types=[
        pltpu.VMEM((C,), jnp.int32),
        pltpu.VMEM((C,), jnp.float32),
        pltpu.VMEM_SHARED((NPAD,), jnp.float32),
    ],
)
def _deg_kernel(dst_hbm, ones_hbm, zn_hbm, out_hbm, dst_v, ones_v, deg_sh):
    c = lax.axis_index("c")
    s = lax.axis_index("s")
    wid = s * 2 + c
    pltpu.sync_copy(zn_hbm.at[pl.ds(s * STRIPE, STRIPE)],
                    deg_sh.at[pl.ds(s * STRIPE, STRIPE)])
    pltpu.sync_copy(ones_hbm, ones_v)
    plsc.subcore_barrier()

    def body(j, carry):
        pltpu.sync_copy(dst_hbm.at[wid, j], dst_v)
        pltpu.sync_copy(ones_v, deg_sh.at[dst_v], add=True)
        return carry

    lax.fori_loop(0, CHUNKS, body, 0)
    plsc.subcore_barrier()
    pltpu.sync_copy(deg_sh.at[pl.ds(s * STRIPE, STRIPE)],
                    out_hbm.at[c, pl.ds(s * STRIPE, STRIPE)])


@functools.partial(
    pl.kernel,
    out_type=jax.ShapeDtypeStruct((2, NPAD, D), jnp.float32),
    mesh=_mesh,
    scratch_types=[
        pltpu.VMEM((C,), jnp.int32),
        pltpu.VMEM((C,), jnp.int32),
        pltpu.VMEM((C, D), jnp.float32),
        pltpu.VMEM_SHARED((NPAD, D), jnp.float32),
        pltpu.SemaphoreType.DMA,
    ],
)
def _edge_kernel(g_hbm, src_hbm, dst_hbm, znd_hbm, out_hbm,
                 src_v, dst_v, rows_v, accum_sh, sem):
    c = lax.axis_index("c")
    s = lax.axis_index("s")
    wid = s * 2 + c
    pltpu.sync_copy(znd_hbm.at[pl.ds(s * STRIPE, STRIPE)],
                    accum_sh.at[pl.ds(s * STRIPE, STRIPE)])
    plsc.subcore_barrier()

    def body(j, carry):
        pltpu.sync_copy(src_hbm.at[wid, j], src_v)
        pltpu.sync_copy(dst_hbm.at[wid, j], dst_v)
        pltpu.async_copy(g_hbm.at[src_v], rows_v, sem).wait()
        pltpu.sync_copy(rows_v, accum_sh.at[dst_v], add=True)
        return carry

    lax.fori_loop(0, CHUNKS, body, 0)
    plsc.subcore_barrier()
    pltpu.sync_copy(accum_sh.at[pl.ds(s * STRIPE, STRIPE)],
                    out_hbm.at[c, pl.ds(s * STRIPE, STRIPE)])


BR = 512


def _k1_body(degT_ref, x_ref, w_ref, dinv_ref, g_ref):
    deg = degT_ref[:, 0:1] + degT_ref[:, 1:2] + 1.0
    dinv = lax.rsqrt(deg)
    dinv_ref[...] = dinv
    h = jnp.dot(x_ref[...], w_ref[...], preferred_element_type=jnp.float32)
    g_ref[...] = h * dinv


def _k1(degT, x_pad, w):
    return pl.pallas_call(
        _k1_body,
        grid=(NPAD // BR,),
        in_specs=[
            pl.BlockSpec((BR, 2), lambda i: (i, 0)),
            pl.BlockSpec((BR, D), lambda i: (i, 0)),
            pl.BlockSpec((D, D), lambda i: (0, 0)),
        ],
        out_specs=[
            pl.BlockSpec((BR, 1), lambda i: (i, 0)),
            pl.BlockSpec((BR, D), lambda i: (i, 0)),
        ],
        out_shape=[
            jax.ShapeDtypeStruct((NPAD, 1), jnp.float32),
            jax.ShapeDtypeStruct((NPAD, D), jnp.float32),
        ],
    )(degT, x_pad, w)


def _k2_body(p_ref, g_ref, dinv_ref, b_ref, w_ref, gout_ref):
    a = p_ref[0] + p_ref[1] + g_ref[...]
    y = jnp.maximum(a * dinv_ref[...] + b_ref[...], 0.0)
    gout_ref[...] = jnp.dot(y, w_ref[...],
                            preferred_element_type=jnp.float32) * dinv_ref[...]


def _k2(p, g, dinv, b, w):
    return pl.pallas_call(
        _k2_body,
        grid=(NPAD // BR,),
        in_specs=[
            pl.BlockSpec((2, BR, D), lambda i: (0, i, 0)),
            pl.BlockSpec((BR, D), lambda i: (i, 0)),
            pl.BlockSpec((BR, 1), lambda i: (i, 0)),
            pl.BlockSpec((1, D), lambda i: (0, 0)),
            pl.BlockSpec((D, D), lambda i: (0, 0)),
        ],
        out_specs=pl.BlockSpec((BR, D), lambda i: (i, 0)),
        out_shape=jax.ShapeDtypeStruct((NPAD, D), jnp.float32),
    )(p, g, dinv, b, w)


def _k3_body(p_ref, g_ref, dinv_ref, b_ref, y_ref):
    a = p_ref[0] + p_ref[1] + g_ref[...]
    y_ref[...] = jnp.maximum(a * dinv_ref[...] + b_ref[...], 0.0)


def _k3(p, g, dinv, b):
    return pl.pallas_call(
        _k3_body,
        grid=(NPAD // BR,),
        in_specs=[
            pl.BlockSpec((2, BR, D), lambda i: (0, i, 0)),
            pl.BlockSpec((BR, D), lambda i: (i, 0)),
            pl.BlockSpec((BR, 1), lambda i: (i, 0)),
            pl.BlockSpec((1, D), lambda i: (0, 0)),
        ],
        out_specs=pl.BlockSpec((BR, D), lambda i: (i, 0)),
        out_shape=jax.ShapeDtypeStruct((NPAD, D), jnp.float32),
    )(p, g, dinv, b)


def kernel(x, edge_index, W1, b1, W2, b2):
    src = edge_index[0]
    dst = edge_index[1]
    pad = EPAD - E
    fill = jnp.full((pad,), N, jnp.int32)
    srcp = jnp.concatenate([src, fill]).reshape(NW, CHUNKS, C)
    dstp = jnp.concatenate([dst, fill]).reshape(NW, CHUNKS, C)
    x_pad = jnp.pad(x, ((0, NPAD - N), (0, 0)))
    zeros_nd = jnp.zeros((NPAD, D), jnp.float32)
    zeros_n = jnp.zeros((NPAD,), jnp.float32)
    ones_c = jnp.ones((C,), jnp.float32)

    degp = _deg_kernel(dstp, ones_c, zeros_n)
    degT = degp.T
    b1r = b1.reshape(1, D)
    b2r = b2.reshape(1, D)

    dinv, g1 = _k1(degT, x_pad, W1)
    p1 = _edge_kernel(g1, srcp, dstp, zeros_nd)
    g2 = _k2(p1, g1, dinv, b1r, W2)
    p2 = _edge_kernel(g2, srcp, dstp, zeros_nd)
    y = _k3(p2, g2, dinv, b2r)
    return y[:N]
